# baseline (device time: 91888 ns/iter reference)
import jax
import jax.numpy as jnp
from jax import lax
from jax.experimental import pallas as pl
from jax.experimental.pallas import tpu as pltpu

N_DEV = 8
COMM_DTYPE = jnp.bfloat16
GROUP_COLS = ((0, 640), (640, 1280), (1280, 2048))
DIM_MASKS = (1, 3, 4)


def kernel(x, w_mat):
    m_total, k_loc = x.shape
    _, n = w_mat.shape
    m_blk = m_total // N_DEV

    def body(x_ref, w_ref, out_ref, *rest):
        acc = rest[0:3]
        st_send0 = rest[3:6]
        st_recv0 = rest[6:9]
        st_recv1 = rest[9:12]
        st_recv2 = rest[12:15]
        (
            x_bf,
            w_bf,
            amax_src,
            amax_ref,
            send_sems,
            recv_sems,
            amax_send_sems,
            amax_recv_sems,
        ) = rest[15:]

        my = lax.axis_index("i")

        amax_ref[:, :] = jnp.zeros((N_DEV, 128), jnp.float32)

        barrier_sem = pltpu.get_barrier_semaphore()
        for o in range(1, N_DEV):
            pl.semaphore_signal(
                barrier_sem,
                inc=1,
                device_id=(lax.rem(my + o, N_DEV),),
                device_id_type=pl.DeviceIdType.MESH,
            )
        pl.semaphore_wait(barrier_sem, N_DEV - 1)

        with jax.named_scope("ph_cast"):
            x_bf[:, :] = x_ref[:, :].astype(jnp.bfloat16)
            w_bf[:, :] = w_ref[:, :].astype(jnp.bfloat16)

        def partial(o, c0, c1):
            return jnp.dot(
                x_bf[pl.ds(o * m_blk, m_blk), :],
                w_bf[:, c0:c1],
                preferred_element_type=jnp.float32,
            )

        def slab(ref, j, nrows=1):
            return ref.at[pl.ds(j * m_blk, nrows * m_blk), :]

        def mk(src, dst, sem_idx, partner):
            return pltpu.make_async_remote_copy(
                src_ref=src,
                dst_ref=dst,
                send_sem=send_sems.at[sem_idx],
                recv_sem=recv_sems.at[sem_idx],
                device_id=(partner,),
                device_id_type=pl.DeviceIdType.MESH,
            )

        CUR = [DIM_MASKS[g % 3] for g in range(3)]
        MA = [DIM_MASKS[(g + 1) % 3] for g in range(3)]
        MB = [DIM_MASKS[(g + 2) % 3] for g in range(3)]
        FS = [[0, MA[g], MB[g], MA[g] ^ MB[g]] for g in range(3)]

        def add_bf16(dst_ref, dj, recv_ref, rj):
            d = pl.ds(dj * m_blk, m_blk)
            r = pl.ds(rj * m_blk, m_blk)
            dst_ref[d, :] = (
                dst_ref[d, :].astype(jnp.float32)
                + recv_ref[r, :].astype(jnp.float32)
            ).astype(COMM_DTYPE)

        rd0 = []
        for g in range(3):
            with jax.named_scope(f"ph_stage0_{g}"):
                c0, c1 = GROUP_COLS[g]
                for j, f in enumerate(FS[g]):
                    st_send0[g][pl.ds(j * m_blk, m_blk), :] = partial(
                        my ^ (CUR[g] ^ f), c0, c1
                    ).astype(COMM_DTYPE)
                r = mk(st_send0[g], st_recv0[g], 4 * g, my ^ CUR[g])
                r.start()
                rd0.append(r)
        with jax.named_scope("ph_keptgemm"):
            for g in range(3):
                c0, c1 = GROUP_COLS[g]
                for j, f in enumerate(FS[g]):
                    acc[g][pl.ds(j * m_blk, m_blk), :] = partial(
                        my ^ f, c0, c1
                    ).astype(COMM_DTYPE)

        rd1 = []
        for g in range(3):
            with jax.named_scope(f"ph_b0_{g}"):
                rd0[g].wait_recv()
                add_bf16(acc[g], 1, st_recv0[g], 1)
                add_bf16(acc[g], 3, st_recv0[g], 3)
                p1 = my ^ MA[g]
                ra = mk(slab(acc[g], 1), slab(st_recv1[g], 0), 4 * g + 1, p1)
                rb = mk(slab(acc[g], 3), slab(st_recv1[g], 1), 4 * g + 2, p1)
                ra.start()
                rb.start()
                rd1.append((ra, rb))
        with jax.named_scope("ph_adds0"):
            for g in range(3):
                add_bf16(acc[g], 0, st_recv0[g], 0)
                add_bf16(acc[g], 2, st_recv0[g], 2)

        rd2 = []
        for g in range(3):
            with jax.named_scope(f"ph_b1_{g}"):
                rd1[g][1].wait_recv()
                add_bf16(acc[g], 2, st_recv1[g], 1)
                r = mk(slab(acc[g], 2), st_recv2[g], 4 * g + 3, my ^ MB[g])
                r.start()
                rd2.append(r)
        with jax.named_scope("ph_adds1"):
            for g in range(3):
                rd1[g][0].wait_recv()
                add_bf16(acc[g], 0, st_recv1[g], 0)

        for g in range(3):
            with jax.named_scope(f"ph_b2_{g}"):
                c0, c1 = GROUP_COLS[g]
                rd2[g].wait_recv()
                out_ref[:, c0:c1] = acc[g][pl.ds(0, m_blk), :].astype(
                    jnp.float32
                ) + st_recv2[g][:, :].astype(jnp.float32)

        with jax.named_scope("ph_drain"):
            for g in range(3):
                rd0[g].wait_send()
                rd1[g][0].wait_send()
                rd1[g][1].wait_send()
                rd2[g].wait_send()

        with jax.named_scope("ph_amax_local"):
            local_amax = jnp.max(jnp.abs(out_ref[:, :]))
        amax_src[:, :] = jnp.full((1, 128), local_amax, jnp.float32)
        ph_xchg = jax.named_scope("ph_amax_xchg")
        ph_xchg.__enter__()
        amax_rd = []
        for o in range(1, N_DEV):
            p = lax.rem(my + o, N_DEV)
            rdma = pltpu.make_async_remote_copy(
                src_ref=amax_src,
                dst_ref=amax_ref.at[pl.ds(my, 1)],
                send_sem=amax_send_sems.at[o],
                recv_sem=amax_recv_sems.at[my],
                device_id=(p,),
                device_id_type=pl.DeviceIdType.MESH,
            )
            rdma.start()
            amax_rd.append(rdma)
        for o in range(1, N_DEV):
            p = lax.rem(my + o, N_DEV)
            recv = pltpu.make_async_remote_copy(
                src_ref=amax_src,
                dst_ref=amax_ref.at[pl.ds(p, 1)],
                send_sem=amax_send_sems.at[o],
                recv_sem=amax_recv_sems.at[p],
                device_id=(p,),
                device_id_type=pl.DeviceIdType.MESH,
            )
            recv.wait_recv()
        for rdma in amax_rd:
            rdma.wait_send()
        ph_xchg.__exit__(None, None, None)

        with jax.named_scope("ph_quant"):
            amax = jnp.maximum(local_amax, jnp.max(amax_ref[:, :]))
            scale = amax / 127.0
            q = jnp.clip(jnp.round(out_ref[:, :] / scale), -127.0, 127.0)
            out_ref[:, :] = q * scale

    widths = [c1 - c0 for c0, c1 in GROUP_COLS]
    stage = lambda rows: [
        pltpu.VMEM((rows * m_blk, w), COMM_DTYPE) for w in widths
    ]

    return pl.pallas_call(
        body,
        out_shape=jax.ShapeDtypeStruct((m_blk, n), jnp.float32),
        in_specs=[
            pl.BlockSpec(memory_space=pltpu.VMEM),
            pl.BlockSpec(memory_space=pltpu.VMEM),
        ],
        out_specs=pl.BlockSpec(memory_space=pltpu.VMEM),
        scratch_shapes=[
            *stage(4),
            *stage(4),
            *stage(4),
            *stage(2),
            *stage(1),
            pltpu.VMEM((m_total, k_loc), jnp.bfloat16),
            pltpu.VMEM((k_loc, n), jnp.bfloat16),
            pltpu.VMEM((1, 128), jnp.float32),
            pltpu.VMEM((N_DEV, 128), jnp.float32),
            pltpu.SemaphoreType.DMA((12,)),
            pltpu.SemaphoreType.DMA((12,)),
            pltpu.SemaphoreType.DMA((N_DEV,)),
            pltpu.SemaphoreType.DMA((N_DEV,)),
        ],
        compiler_params=pltpu.CompilerParams(
            collective_id=0, vmem_limit_bytes=60 * 1024 * 1024
        ),
    )(x, w_mat)


# device time: 79791 ns/iter; 1.1516x vs baseline; 1.1516x over previous
import jax
import jax.numpy as jnp
from jax import lax
from jax.experimental import pallas as pl
from jax.experimental.pallas import tpu as pltpu

N_DEV = 8
COMM_DTYPE = jnp.bfloat16
GROUP_COLS = ((0, 640), (640, 1280), (1280, 2048))
DIM_MASKS = (1, 3, 4)


def kernel(x, w_mat):
    m_total, k_loc = x.shape
    _, n = w_mat.shape
    m_blk = m_total // N_DEV

    def body(x_ref, w_ref, out_ref, *rest):
        acc = rest[0:3]
        st_send0 = rest[3:6]
        st_recv0 = rest[6:9]
        st_recv1 = rest[9:12]
        st_recv2 = rest[12:15]
        (
            x_bf,
            w_bf,
            amax_src,
            amax_ref,
            send_sems,
            recv_sems,
            amax_send_sems,
            amax_recv_sems,
        ) = rest[15:]

        my = lax.axis_index("i")

        amax_ref[:, :] = jnp.zeros((N_DEV, 128), jnp.float32)

        barrier_sem = pltpu.get_barrier_semaphore()
        for o in range(1, N_DEV):
            pl.semaphore_signal(
                barrier_sem,
                inc=1,
                device_id=(lax.rem(my + o, N_DEV),),
                device_id_type=pl.DeviceIdType.MESH,
            )
        pl.semaphore_wait(barrier_sem, N_DEV - 1)

        x_bf[:, :] = x_ref[:, :].astype(jnp.bfloat16)
        w_bf[:, :] = w_ref[:, :].astype(jnp.bfloat16)

        def partial(o, c0, c1):
            return jnp.dot(
                x_bf[pl.ds(o * m_blk, m_blk), :],
                w_bf[:, c0:c1],
                preferred_element_type=jnp.float32,
            )

        def slab(ref, j, nrows=1):
            return ref.at[pl.ds(j * m_blk, nrows * m_blk), :]

        def mk(src, dst, sem_idx, partner):
            return pltpu.make_async_remote_copy(
                src_ref=src,
                dst_ref=dst,
                send_sem=send_sems.at[sem_idx],
                recv_sem=recv_sems.at[sem_idx],
                device_id=(partner,),
                device_id_type=pl.DeviceIdType.MESH,
            )

        CUR = [DIM_MASKS[g % 3] for g in range(3)]
        MA = [DIM_MASKS[(g + 1) % 3] for g in range(3)]
        MB = [DIM_MASKS[(g + 2) % 3] for g in range(3)]
        FS = [[0, MA[g], MB[g], MA[g] ^ MB[g]] for g in range(3)]

        def add_bf16(dst_ref, dj, recv_ref, rj):
            d = pl.ds(dj * m_blk, m_blk)
            r = pl.ds(rj * m_blk, m_blk)
            dst_ref[d, :] = (
                dst_ref[d, :].astype(jnp.float32)
                + recv_ref[r, :].astype(jnp.float32)
            ).astype(COMM_DTYPE)

        SLAB_ORDER = (1, 3, 2, 0)
        rd0 = [[None] * 4 for _ in range(3)]
        for j in SLAB_ORDER:
            for g in range(3):
                c0, c1 = GROUP_COLS[g]
                st_send0[g][pl.ds(j * m_blk, m_blk), :] = partial(
                    my ^ (CUR[g] ^ FS[g][j]), c0, c1
                ).astype(COMM_DTYPE)
                r = mk(
                    slab(st_send0[g], j),
                    slab(st_recv0[g], j),
                    8 * g + j,
                    my ^ CUR[g],
                )
                r.start()
                rd0[g][j] = r
        for j in SLAB_ORDER:
            for g in range(3):
                c0, c1 = GROUP_COLS[g]
                acc[g][pl.ds(j * m_blk, m_blk), :] = partial(
                    my ^ FS[g][j], c0, c1
                ).astype(COMM_DTYPE)

        rd1 = []
        for g in range(3):
            rd0[g][1].wait_recv()
            add_bf16(acc[g], 1, st_recv0[g], 1)
            rd0[g][3].wait_recv()
            add_bf16(acc[g], 3, st_recv0[g], 3)
            p1 = my ^ MA[g]
            rb = mk(slab(acc[g], 3), slab(st_recv1[g], 1), 8 * g + 5, p1)
            ra = mk(slab(acc[g], 1), slab(st_recv1[g], 0), 8 * g + 4, p1)
            rb.start()
            ra.start()
            rd1.append((ra, rb))
        for g in range(3):
            rd0[g][2].wait_recv()
            add_bf16(acc[g], 2, st_recv0[g], 2)

        rd2 = []
        for g in range(3):
            rd1[g][1].wait_recv()
            add_bf16(acc[g], 2, st_recv1[g], 1)
            r = mk(slab(acc[g], 2), st_recv2[g], 8 * g + 6, my ^ MB[g])
            r.start()
            rd2.append(r)
        for g in range(3):
            rd0[g][0].wait_recv()
            add_bf16(acc[g], 0, st_recv0[g], 0)
            rd1[g][0].wait_recv()
            add_bf16(acc[g], 0, st_recv1[g], 0)

        for g in range(3):
            c0, c1 = GROUP_COLS[g]
            rd2[g].wait_recv()
            out_ref[:, c0:c1] = acc[g][pl.ds(0, m_blk), :].astype(
                jnp.float32
            ) + st_recv2[g][:, :].astype(jnp.float32)

        for g in range(3):
            for j in range(4):
                rd0[g][j].wait_send()
            rd1[g][0].wait_send()
            rd1[g][1].wait_send()
            rd2[g].wait_send()

        local_amax = jnp.max(jnp.abs(out_ref[:, :]))
        amax_src[:, :] = jnp.full((1, 128), local_amax, jnp.float32)
        amax_rd = []
        for o in range(1, N_DEV):
            p = lax.rem(my + o, N_DEV)
            rdma = pltpu.make_async_remote_copy(
                src_ref=amax_src,
                dst_ref=amax_ref.at[pl.ds(my, 1)],
                send_sem=amax_send_sems.at[o],
                recv_sem=amax_recv_sems.at[my],
                device_id=(p,),
                device_id_type=pl.DeviceIdType.MESH,
            )
            rdma.start()
            amax_rd.append(rdma)
        for o in range(1, N_DEV):
            p = lax.rem(my + o, N_DEV)
            recv = pltpu.make_async_remote_copy(
                src_ref=amax_src,
                dst_ref=amax_ref.at[pl.ds(p, 1)],
                send_sem=amax_send_sems.at[o],
                recv_sem=amax_recv_sems.at[p],
                device_id=(p,),
                device_id_type=pl.DeviceIdType.MESH,
            )
            recv.wait_recv()
        for rdma in amax_rd:
            rdma.wait_send()

        amax = jnp.maximum(local_amax, jnp.max(amax_ref[:, :]))
        scale = amax / 127.0
        q = jnp.clip(jnp.round(out_ref[:, :] / scale), -127.0, 127.0)
        out_ref[:, :] = q * scale

    widths = [c1 - c0 for c0, c1 in GROUP_COLS]
    stage = lambda rows: [
        pltpu.VMEM((rows * m_blk, w), COMM_DTYPE) for w in widths
    ]

    return pl.pallas_call(
        body,
        out_shape=jax.ShapeDtypeStruct((m_blk, n), jnp.float32),
        in_specs=[
            pl.BlockSpec(memory_space=pltpu.VMEM),
            pl.BlockSpec(memory_space=pltpu.VMEM),
        ],
        out_specs=pl.BlockSpec(memory_space=pltpu.VMEM),
        scratch_shapes=[
            *stage(4),
            *stage(4),
            *stage(4),
            *stage(2),
            *stage(1),
            pltpu.VMEM((m_total, k_loc), jnp.bfloat16),
            pltpu.VMEM((k_loc, n), jnp.bfloat16),
            pltpu.VMEM((1, 128), jnp.float32),
            pltpu.VMEM((N_DEV, 128), jnp.float32),
            pltpu.SemaphoreType.DMA((24,)),
            pltpu.SemaphoreType.DMA((24,)),
            pltpu.SemaphoreType.DMA((N_DEV,)),
            pltpu.SemaphoreType.DMA((N_DEV,)),
        ],
        compiler_params=pltpu.CompilerParams(
            collective_id=0, vmem_limit_bytes=60 * 1024 * 1024
        ),
    )(x, w_mat)
